# Initial kernel scaffold; baseline (speedup 1.0000x reference)
import jax
import jax.numpy as jnp
from jax.experimental import pallas as pl


def kernel(x_fft_sliced):
    raise NotImplementedError("write your pallas kernel here")


# speedup vs baseline: 7.9072x; 7.9072x over previous
import numpy as np
import jax
import jax.numpy as jnp
from jax.experimental import pallas as pl

_N_MODES = (32, 32, 32)
_LMAX = 2
_RADIAL_BINS = 16
_EPS = 1e-06
_NUM_SH = (_LMAX + 1) ** 2


def _sym_k_np(n):
    k = n // 2
    pos = np.arange(k + n % 2, dtype=np.float32)
    neg = np.arange(-k, 0, dtype=np.float32)
    return np.concatenate([pos, neg], axis=0)


def _real_sph_np(coords, lmax, eps):
    x = coords[:, 0]
    y = coords[:, 1]
    z = coords[:, 2]
    r = np.maximum(np.linalg.norm(coords, axis=-1), eps)
    x = x / r
    y = y / r
    z = z / r
    basis = [0.28209479177387814 * np.ones_like(x)]
    if lmax >= 1:
        basis.extend([0.4886025119029199 * y, 0.4886025119029199 * z,
                      0.4886025119029199 * x])
    if lmax >= 2:
        basis.extend([
            1.0925484305920792 * x * y,
            1.0925484305920792 * y * z,
            0.31539156525252005 * (3.0 * z * z - 1.0),
            1.0925484305920792 * x * z,
            0.5462742152960396 * (x * x - y * y),
        ])
    basis = np.stack(basis, axis=-1)
    zero_mask = np.abs(coords).sum(axis=-1) < eps
    if zero_mask.any() and basis.shape[1] > 1:
        basis = basis.copy()
        basis[zero_mask, 1:] = 0.0
    return basis


def _build_weight_tables():
    kx = _sym_k_np(_N_MODES[0])
    ky = _sym_k_np(_N_MODES[1])
    kz = _sym_k_np(_N_MODES[2])
    KX, KY, KZ = np.meshgrid(kx, ky, kz, indexing='ij')
    coords = np.stack([KX, KY, KZ], axis=-1).reshape(-1, 3)
    radii = np.linalg.norm(coords, axis=-1)
    max_r = max(float(radii.max()), 1.0)
    bin_edges = np.linspace(0.0, max_r + 1e-06, _RADIAL_BINS + 1)
    shell_ids = np.searchsorted(bin_edges[1:-1], radii, side='left')

    n_total = coords.shape[0]
    shells = []
    for sid in range(_RADIAL_BINS):
        idx = np.nonzero(shell_ids == sid)[0]
        if idx.size == 0:
            continue
        basis = _real_sph_np(coords[idx], _LMAX, _EPS).astype(np.float32)
        pinv = np.linalg.pinv(basis).astype(np.float32)
        shells.append((idx, basis, pinv))

    n_shells = len(shells)
    k_dim = n_shells * _NUM_SH
    w_full = np.zeros((n_total, k_dim), dtype=np.float32)
    b_full = np.zeros((n_total, k_dim), dtype=np.float32)
    for s, (idx, basis, pinv) in enumerate(shells):
        w_full[idx, s * _NUM_SH:(s + 1) * _NUM_SH] = pinv.T
        b_full[idx, s * _NUM_SH:(s + 1) * _NUM_SH] = basis
    return w_full, b_full


_W_FULL_NP, _B_FULL_NP = _build_weight_tables()
_N_FLAT = _W_FULL_NP.shape[0]
_K_DIM = _W_FULL_NP.shape[1]
_P_BLOCK = 2048


def _project_body(x_ref, w_ref, coeff_ref):
    @pl.when(pl.program_id(0) == 0)
    def _():
        coeff_ref[...] = jnp.zeros_like(coeff_ref)

    coeff_ref[...] += jax.lax.dot_general(
        x_ref[...], w_ref[...], (((1,), (0,)), ((), ())),
        preferred_element_type=jnp.float32)


def _reconstruct_body(coeff_ref, b_ref, out_ref):
    out_ref[...] = jax.lax.dot_general(
        coeff_ref[...], b_ref[...], (((1,), (1,)), ((), ())),
        preferred_element_type=jnp.float32)


def kernel(x_fft_sliced):
    b, c = x_fft_sliced.shape[:2]
    rows = b * c
    flat = x_fft_sliced.reshape(rows, _N_FLAT)
    w_full = jnp.asarray(_W_FULL_NP)
    b_full = jnp.asarray(_B_FULL_NP)
    n_blocks = _N_FLAT // _P_BLOCK

    coeff = pl.pallas_call(
        _project_body,
        grid=(n_blocks,),
        in_specs=[
            pl.BlockSpec((rows, _P_BLOCK), lambda i: (0, i)),
            pl.BlockSpec((_P_BLOCK, _K_DIM), lambda i: (i, 0)),
        ],
        out_specs=pl.BlockSpec((rows, _K_DIM), lambda i: (0, 0)),
        out_shape=jax.ShapeDtypeStruct((rows, _K_DIM), jnp.float32),
    )(flat, w_full)

    out = pl.pallas_call(
        _reconstruct_body,
        grid=(n_blocks,),
        in_specs=[
            pl.BlockSpec((rows, _K_DIM), lambda i: (0, 0)),
            pl.BlockSpec((_P_BLOCK, _K_DIM), lambda i: (i, 0)),
        ],
        out_specs=pl.BlockSpec((rows, _P_BLOCK), lambda i: (0, i)),
        out_shape=jax.ShapeDtypeStruct((rows, _N_FLAT), jnp.float32),
    )(coeff, b_full)

    return out.reshape(x_fft_sliced.shape)
